# baseline (speedup 1.0000x reference)
import functools

import jax
import jax.numpy as jnp
from jax import lax
from jax.experimental import pallas as pl
from jax.experimental.pallas import tpu as pltpu
from jax.experimental.pallas import tpu_sc as plsc

N_NODES = 10000
N_EDGES = 320000
D_FEAT = 128
N_HIDDEN = 128
N_CLASSES = 40
C_PAD = 64

NC = 2
NS = 16
NW = NC * NS

E2 = 2 * N_EDGES
BLK = 120
NBLK = 174
EPT = NBLK * BLK
EPAD = NW * EPT
NPAD = 10240
RPT = NPAD // NS

_MESH = plsc.VectorSubcoreMesh(core_axis_name="c", subcore_axis_name="s")
_SC_PARAMS = pltpu.CompilerParams(use_tc_tiling_on_sc=False)

NSLOT = 6
PF = 4
DSLOT = NSLOT


@functools.partial(
    pl.kernel,
    mesh=_MESH,
    out_type=jax.ShapeDtypeStruct((NC, NPAD, 16), jnp.float32),
    scratch_types=[
        pltpu.VMEM((NBLK, BLK), jnp.int32),
        pltpu.VMEM((BLK, 16), jnp.float32),
        pltpu.VMEM_SHARED((NPAD, 16), jnp.float32),
        pltpu.SemaphoreType.DMA((DSLOT,)),
    ],
    compiler_params=_SC_PARAMS,
)
def _deg_kernel(gd_hbm, ones_hbm, zeros_hbm, out_hbm, gd_v, ones_v, acc, ssem):
    cid = lax.axis_index("c")
    sid = lax.axis_index("s")
    wid = cid * NS + sid
    pltpu.sync_copy(gd_hbm.at[wid], gd_v)
    pltpu.sync_copy(ones_hbm, ones_v)
    pltpu.sync_copy(zeros_hbm.at[pl.ds(sid * RPT, RPT)], acc.at[pl.ds(sid * RPT, RPT)])
    plsc.subcore_barrier()

    def s_start(j, b):
        pltpu.async_copy(ones_v, acc.at[gd_v.at[j]], ssem.at[b], add=True)

    def s_wait(j, b):
        pltpu.make_async_copy(ones_v, acc.at[gd_v.at[j]], ssem.at[b]).wait()

    for k in range(DSLOT):
        s_start(k, k)

    @pl.loop(0, NBLK - DSLOT, step=DSLOT)
    def _(g):
        for k in range(DSLOT):
            s_wait(g + k, k)
            s_start(g + k + DSLOT, k)

    for k in range(DSLOT):
        s_wait(NBLK - DSLOT + k, k)

    plsc.subcore_barrier()
    pltpu.sync_copy(acc.at[pl.ds(sid * RPT, RPT)],
                    out_hbm.at[cid, pl.ds(sid * RPT, RPT)])


def _make_agg_kernel(width):
    @functools.partial(
        pl.kernel,
        mesh=_MESH,
        out_type=jax.ShapeDtypeStruct((NC, NPAD, width), jnp.float32),
        scratch_types=(
            [pltpu.VMEM((NBLK, BLK), jnp.int32),
             pltpu.VMEM((NBLK, BLK), jnp.int32)]
            + [pltpu.VMEM((BLK, width), jnp.float32) for _ in range(NSLOT)]
            + [pltpu.VMEM_SHARED((NPAD, width), jnp.float32),
               pltpu.SemaphoreType.DMA((NSLOT,)),
               pltpu.SemaphoreType.DMA((NSLOT,))]
        ),
        compiler_params=_SC_PARAMS,
    )
    def agg(gs_hbm, gd_hbm, h_hbm, zeros_hbm, out_hbm, gs_v, gd_v, *rest):
        rows = rest[:NSLOT]
        acc, gsem, ssem = rest[NSLOT:]
        cid = lax.axis_index("c")
        sid = lax.axis_index("s")
        wid = cid * NS + sid
        pltpu.sync_copy(gs_hbm.at[wid], gs_v)
        pltpu.sync_copy(gd_hbm.at[wid], gd_v)
        pltpu.sync_copy(zeros_hbm.at[pl.ds(sid * RPT, RPT)],
                        acc.at[pl.ds(sid * RPT, RPT)])
        plsc.subcore_barrier()

        def g_start(j, b):
            pltpu.async_copy(h_hbm.at[gs_v.at[j]], rows[b], gsem.at[b])

        def g_wait(j, b):
            pltpu.make_async_copy(h_hbm.at[gs_v.at[j]], rows[b],
                                  gsem.at[b]).wait()

        def s_start(j, b):
            pltpu.async_copy(rows[b], acc.at[gd_v.at[j]], ssem.at[b],
                             add=True)

        def s_wait(j, b):
            pltpu.make_async_copy(rows[b], acc.at[gd_v.at[j]],
                                  ssem.at[b]).wait()

        for j in range(PF):
            g_start(j, j % NSLOT)
        for j in range(NSLOT - PF):
            g_wait(j, j % NSLOT)
            s_start(j, j % NSLOT)
            g_start(j + PF, (j + PF) % NSLOT)

        j0 = NSLOT - PF
        @pl.loop(j0, NBLK - PF, step=NSLOT)
        def _(g):
            for k in range(NSLOT):
                bj = (j0 + k) % NSLOT
                bw = k % NSLOT
                j = g + k
                s_wait(j - j0, bw)
                g_wait(j, bj)
                s_start(j, bj)
                g_start(j + PF, bw)

        for j in range(NBLK - PF, NBLK):
            s_wait(j - j0, (j - j0) % NSLOT)
            g_wait(j, j % NSLOT)
            s_start(j, j % NSLOT)
        for j in range(NBLK - j0, NBLK):
            s_wait(j, j % NSLOT)

        plsc.subcore_barrier()
        pltpu.sync_copy(acc.at[pl.ds(sid * RPT, RPT)],
                        out_hbm.at[cid, pl.ds(sid * RPT, RPT)])

    return agg


HALF = D_FEAT // 2
_agg64 = _make_agg_kernel(HALF)


_RB = 200


def _normscale_body(x_ref, d0_ref, d1_ref, ha_ref, hb_ref, nb_ref):
    deg = d0_ref[:, 0:1] + d1_ref[:, 0:1] + 1.0
    norm = lax.rsqrt(deg)
    nb = jnp.broadcast_to(norm, (_RB, D_FEAT))
    nb_ref[...] = nb
    h0 = x_ref[...] * nb
    ha_ref[...] = h0[:, :HALF]
    hb_ref[...] = h0[:, HALF:]


def _layer1_body(a0a_ref, a1a_ref, a0b_ref, a1b_ref, ha_ref, hb_ref, nb_ref,
                 w1a_ref, w1b_ref, b1_ref, w2_ref, h2_ref):
    nb = nb_ref[...]
    agg_a = a0a_ref[...] + a1a_ref[...] + ha_ref[...]
    agg_b = a0b_ref[...] + a1b_ref[...] + hb_ref[...]
    z = (jnp.dot(agg_a, w1a_ref[...], preferred_element_type=jnp.float32)
         + jnp.dot(agg_b, w1b_ref[...], preferred_element_type=jnp.float32))
    h1 = jnp.maximum(z * nb + b1_ref[...], 0.0)
    h2_ref[...] = jnp.dot(h1 * nb, w2_ref[...], preferred_element_type=jnp.float32)


def _out_body(e0_ref, e1_ref, h2_ref, n48_ref, b2_ref, o_ref):
    v = (e0_ref[...] + e1_ref[...] + h2_ref[...]) * n48_ref[...] + b2_ref[...]
    mask = lax.broadcasted_iota(jnp.int32, (_RB, C_PAD), 1) < N_CLASSES
    vm = jnp.where(mask, v, -jnp.inf)
    m = jnp.max(vm, axis=1, keepdims=True)
    ex = jnp.where(mask, jnp.exp(v - m), 0.0)
    s = jnp.sum(ex, axis=1, keepdims=True)
    o_ref[...] = v - m - jnp.log(s)


def _row_spec(width):
    return pl.BlockSpec((_RB, width), lambda i: (i, 0))


def _full_spec(r, c):
    return pl.BlockSpec((r, c), lambda i: (0, 0))


def kernel(x, edge_index, W1, b1, W2, b2):
    src = edge_index[0]
    dst = edge_index[1]
    npad = EPAD - E2
    gs = jnp.concatenate([src, dst, jnp.zeros((npad,), jnp.int32)])
    gd = jnp.concatenate([dst, src, jnp.full((npad,), N_NODES, jnp.int32)])
    gs = gs.reshape(NW, NBLK, BLK)
    gd = gd.reshape(NW, NBLK, BLK)

    ones16 = jnp.ones((BLK, 16), jnp.float32)
    zeros16 = jnp.zeros((NPAD, 16), jnp.float32)
    zeros64 = jnp.zeros((NPAD, HALF), jnp.float32)

    degp = _deg_kernel(gd, ones16, zeros16)

    grid = N_NODES // _RB
    ha, hb, nb = pl.pallas_call(
        _normscale_body,
        grid=(grid,),
        in_specs=[_row_spec(D_FEAT), _row_spec(16), _row_spec(16)],
        out_specs=[_row_spec(HALF), _row_spec(HALF), _row_spec(D_FEAT)],
        out_shape=[
            jax.ShapeDtypeStruct((N_NODES, HALF), jnp.float32),
            jax.ShapeDtypeStruct((N_NODES, HALF), jnp.float32),
            jax.ShapeDtypeStruct((N_NODES, D_FEAT), jnp.float32),
        ],
    )(x, degp[0, :N_NODES], degp[1, :N_NODES])

    aggpa = _agg64(gs, gd, ha, zeros64)
    aggpb = _agg64(gs, gd, hb, zeros64)

    w2p = jnp.concatenate(
        [W2, jnp.zeros((N_HIDDEN, C_PAD - N_CLASSES), jnp.float32)], axis=1)
    h2 = pl.pallas_call(
        _layer1_body,
        grid=(grid,),
        in_specs=[
            _row_spec(HALF), _row_spec(HALF), _row_spec(HALF), _row_spec(HALF),
            _row_spec(HALF), _row_spec(HALF), _row_spec(D_FEAT),
            _full_spec(HALF, N_HIDDEN), _full_spec(HALF, N_HIDDEN),
            _full_spec(1, N_HIDDEN), _full_spec(N_HIDDEN, C_PAD),
        ],
        out_specs=_row_spec(C_PAD),
        out_shape=jax.ShapeDtypeStruct((N_NODES, C_PAD), jnp.float32),
    )(aggpa[0, :N_NODES], aggpa[1, :N_NODES],
      aggpb[0, :N_NODES], aggpb[1, :N_NODES], ha, hb, nb,
      W1[:HALF], W1[HALF:], b1.reshape(1, N_HIDDEN), w2p)

    agg2p = _agg64(gs, gd, h2, zeros64)

    b2p = jnp.concatenate(
        [b2, jnp.zeros((C_PAD - N_CLASSES,), jnp.float32)]).reshape(1, C_PAD)
    out48 = pl.pallas_call(
        _out_body,
        grid=(grid,),
        in_specs=[
            _row_spec(C_PAD), _row_spec(C_PAD), _row_spec(C_PAD),
            _row_spec(C_PAD), _full_spec(1, C_PAD),
        ],
        out_specs=_row_spec(C_PAD),
        out_shape=jax.ShapeDtypeStruct((N_NODES, C_PAD), jnp.float32),
    )(agg2p[0, :N_NODES], agg2p[1, :N_NODES], h2, nb[:, :C_PAD], b2p)

    return out48[:, :N_CLASSES]


# speedup vs baseline: 2.9372x; 2.9372x over previous
import functools

import jax
import jax.numpy as jnp
from jax import lax
from jax.experimental import pallas as pl
from jax.experimental.pallas import tpu as pltpu
from jax.experimental.pallas import tpu_sc as plsc

N_NODES = 10000
N_EDGES = 320000
D_FEAT = 128
N_HIDDEN = 128
N_CLASSES = 40
C_PAD = 64

NC = 2
NS = 16
NW = NC * NS

E2 = 2 * N_EDGES
BLK = 128
NBLK_F = 255
NBLK_S = 60
FAST_CID = 0
NBLKMAX = NBLK_F
NPAD = 10240
RPT = NPAD // NS

_MESH = plsc.VectorSubcoreMesh(core_axis_name="c", subcore_axis_name="s")
_SC_PARAMS = pltpu.CompilerParams(use_tc_tiling_on_sc=False)

NSLOT = 3
PF = 2
DSLOT = NSLOT


@functools.partial(
    pl.kernel,
    mesh=_MESH,
    out_type=jax.ShapeDtypeStruct((NC, NPAD, 16), jnp.float32),
    scratch_types=[
        pltpu.VMEM((NBLKMAX, BLK), jnp.int32),
        pltpu.VMEM((BLK, 16), jnp.float32),
        pltpu.VMEM_SHARED((NPAD, 16), jnp.float32),
        pltpu.SemaphoreType.DMA((DSLOT,)),
    ],
    compiler_params=_SC_PARAMS,
)
def _deg_kernel(gd_hbm, ones_hbm, zeros_hbm, out_hbm, gd_v, ones_v, acc, ssem):
    cid = lax.axis_index("c")
    sid = lax.axis_index("s")
    wid = cid * NS + sid
    nblk = jnp.where(cid == FAST_CID, NBLK_F, NBLK_S)
    pltpu.sync_copy(gd_hbm.at[wid], gd_v)
    pltpu.sync_copy(ones_hbm, ones_v)
    pltpu.sync_copy(zeros_hbm.at[pl.ds(sid * RPT, RPT)], acc.at[pl.ds(sid * RPT, RPT)])
    plsc.subcore_barrier()

    def s_start(j, b):
        pltpu.async_copy(ones_v, acc.at[gd_v.at[j]], ssem.at[b], add=True)

    def s_wait(j, b):
        pltpu.make_async_copy(ones_v, acc.at[gd_v.at[j]], ssem.at[b]).wait()

    for k in range(DSLOT):
        s_start(k, k)

    @pl.loop(0, nblk - DSLOT, step=DSLOT)
    def _(g):
        for k in range(DSLOT):
            s_wait(g + k, k)
            s_start(g + k + DSLOT, k)

    for k in range(DSLOT):
        s_wait(nblk - DSLOT + k, k)

    plsc.subcore_barrier()
    pltpu.sync_copy(acc.at[pl.ds(sid * RPT, RPT)],
                    out_hbm.at[cid, pl.ds(sid * RPT, RPT)])


def _make_agg_kernel(width):
    @functools.partial(
        pl.kernel,
        mesh=_MESH,
        out_type=jax.ShapeDtypeStruct((NC, NPAD, width), jnp.float32),
        scratch_types=(
            [pltpu.VMEM((NBLKMAX, BLK), jnp.int32),
             pltpu.VMEM((NBLKMAX, BLK), jnp.int32)]
            + [pltpu.VMEM((BLK, width), jnp.float32) for _ in range(NSLOT)]
            + [pltpu.VMEM_SHARED((NPAD, width), jnp.float32),
               pltpu.SemaphoreType.DMA((NSLOT,)),
               pltpu.SemaphoreType.DMA((NSLOT,))]
        ),
        compiler_params=_SC_PARAMS,
    )
    def agg(gs_hbm, gd_hbm, h_hbm, zeros_hbm, out_hbm, gs_v, gd_v, *rest):
        rows = rest[:NSLOT]
        acc, gsem, ssem = rest[NSLOT:]
        cid = lax.axis_index("c")
        sid = lax.axis_index("s")
        wid = cid * NS + sid
        nblk = jnp.where(cid == FAST_CID, NBLK_F, NBLK_S)
        pltpu.sync_copy(gs_hbm.at[wid], gs_v)
        pltpu.sync_copy(gd_hbm.at[wid], gd_v)
        pltpu.sync_copy(zeros_hbm.at[pl.ds(sid * RPT, RPT)],
                        acc.at[pl.ds(sid * RPT, RPT)])
        plsc.subcore_barrier()

        def g_start(j, b):
            pltpu.async_copy(h_hbm.at[gs_v.at[j]], rows[b], gsem.at[b])

        def g_wait(j, b):
            pltpu.make_async_copy(h_hbm.at[gs_v.at[j]], rows[b],
                                  gsem.at[b]).wait()

        def s_start(j, b):
            pltpu.async_copy(rows[b], acc.at[gd_v.at[j]], ssem.at[b],
                             add=True)

        def s_wait(j, b):
            pltpu.make_async_copy(rows[b], acc.at[gd_v.at[j]],
                                  ssem.at[b]).wait()

        j0 = NSLOT - PF
        for j in range(PF):
            g_start(j, j % NSLOT)
        for j in range(j0):
            g_wait(j, j % NSLOT)
            s_start(j, j % NSLOT)
            g_start(j + PF, (j + PF) % NSLOT)

        @pl.loop(j0, nblk - PF, step=NSLOT)
        def _(g):
            for k in range(NSLOT):
                bj = (j0 + k) % NSLOT
                bw = k % NSLOT
                j = g + k
                s_wait(j - j0, bw)
                g_wait(j, bj)
                s_start(j, bj)
                g_start(j + PF, bw)

        for i in range(PF):
            j = nblk - PF + i
            s_wait(j - j0, i % NSLOT)
            g_wait(j, (j0 + i) % NSLOT)
            s_start(j, (j0 + i) % NSLOT)
        for i in range(j0):
            s_wait(nblk - j0 + i, (PF + i) % NSLOT)

        plsc.subcore_barrier()
        pltpu.sync_copy(acc.at[pl.ds(sid * RPT, RPT)],
                        out_hbm.at[cid, pl.ds(sid * RPT, RPT)])

    return agg


HALF = D_FEAT // 2
_agg64 = _make_agg_kernel(HALF)


_RB = 200


def _normscale_body(x_ref, d0_ref, d1_ref, ha_ref, hb_ref, nb_ref):
    deg = d0_ref[:, 0:1] + d1_ref[:, 0:1] + 1.0
    norm = lax.rsqrt(deg)
    nb = jnp.broadcast_to(norm, (_RB, D_FEAT))
    nb_ref[...] = nb
    h0 = x_ref[...] * nb
    ha_ref[...] = h0[:, :HALF]
    hb_ref[...] = h0[:, HALF:]


def _layer1_body(a0a_ref, a1a_ref, a0b_ref, a1b_ref, ha_ref, hb_ref, nb_ref,
                 w1a_ref, w1b_ref, b1_ref, w2_ref, h2_ref):
    nb = nb_ref[...]
    agg_a = a0a_ref[...] + a1a_ref[...] + ha_ref[...]
    agg_b = a0b_ref[...] + a1b_ref[...] + hb_ref[...]
    z = (jnp.dot(agg_a, w1a_ref[...], preferred_element_type=jnp.float32)
         + jnp.dot(agg_b, w1b_ref[...], preferred_element_type=jnp.float32))
    h1 = jnp.maximum(z * nb + b1_ref[...], 0.0)
    h2_ref[...] = jnp.dot(h1 * nb, w2_ref[...], preferred_element_type=jnp.float32)


def _out_body(e0_ref, e1_ref, h2_ref, n48_ref, b2_ref, o_ref):
    v = (e0_ref[...] + e1_ref[...] + h2_ref[...]) * n48_ref[...] + b2_ref[...]
    mask = lax.broadcasted_iota(jnp.int32, (_RB, C_PAD), 1) < N_CLASSES
    vm = jnp.where(mask, v, -jnp.inf)
    m = jnp.max(vm, axis=1, keepdims=True)
    ex = jnp.where(mask, jnp.exp(v - m), 0.0)
    s = jnp.sum(ex, axis=1, keepdims=True)
    o_ref[...] = v - m - jnp.log(s)


def _row_spec(width):
    return pl.BlockSpec((_RB, width), lambda i: (i, 0))


def _full_spec(r, c):
    return pl.BlockSpec((r, c), lambda i: (0, 0))


def kernel(x, edge_index, W1, b1, W2, b2):
    src = edge_index[0]
    dst = edge_index[1]
    gs_flat = jnp.concatenate([src, dst])
    gd_flat = jnp.concatenate([dst, src])
    cap_f = NS * NBLK_F * BLK
    cap_s = NS * NBLK_S * BLK
    pad_s = cap_s - (E2 - cap_f)
    gs_f = gs_flat[:cap_f].reshape(NS, NBLK_F, BLK)
    gd_f = gd_flat[:cap_f].reshape(NS, NBLK_F, BLK)
    gs_s = jnp.concatenate(
        [gs_flat[cap_f:], jnp.zeros((pad_s,), jnp.int32)]).reshape(NS, NBLK_S, BLK)
    gd_s = jnp.concatenate(
        [gd_flat[cap_f:], jnp.full((pad_s,), N_NODES, jnp.int32)]).reshape(NS, NBLK_S, BLK)
    gs_s = jnp.pad(gs_s, ((0, 0), (0, NBLKMAX - NBLK_S), (0, 0)))
    gd_s = jnp.pad(gd_s, ((0, 0), (0, NBLKMAX - NBLK_S), (0, 0)),
                   constant_values=N_NODES)
    if FAST_CID == 0:
        gs = jnp.concatenate([gs_f, gs_s])
        gd = jnp.concatenate([gd_f, gd_s])
    else:
        gs = jnp.concatenate([gs_s, gs_f])
        gd = jnp.concatenate([gd_s, gd_f])

    ones16 = jnp.ones((BLK, 16), jnp.float32)
    zeros16 = jnp.zeros((NPAD, 16), jnp.float32)
    zeros64 = jnp.zeros((NPAD, HALF), jnp.float32)

    degp = _deg_kernel(gd, ones16, zeros16)

    grid = N_NODES // _RB
    ha, hb, nb = pl.pallas_call(
        _normscale_body,
        grid=(grid,),
        in_specs=[_row_spec(D_FEAT), _row_spec(16), _row_spec(16)],
        out_specs=[_row_spec(HALF), _row_spec(HALF), _row_spec(D_FEAT)],
        out_shape=[
            jax.ShapeDtypeStruct((N_NODES, HALF), jnp.float32),
            jax.ShapeDtypeStruct((N_NODES, HALF), jnp.float32),
            jax.ShapeDtypeStruct((N_NODES, D_FEAT), jnp.float32),
        ],
    )(x, degp[0, :N_NODES], degp[1, :N_NODES])

    aggpa = _agg64(gs, gd, ha, zeros64)
    aggpb = _agg64(gs, gd, hb, zeros64)

    w2p = jnp.concatenate(
        [W2, jnp.zeros((N_HIDDEN, C_PAD - N_CLASSES), jnp.float32)], axis=1)
    h2 = pl.pallas_call(
        _layer1_body,
        grid=(grid,),
        in_specs=[
            _row_spec(HALF), _row_spec(HALF), _row_spec(HALF), _row_spec(HALF),
            _row_spec(HALF), _row_spec(HALF), _row_spec(D_FEAT),
            _full_spec(HALF, N_HIDDEN), _full_spec(HALF, N_HIDDEN),
            _full_spec(1, N_HIDDEN), _full_spec(N_HIDDEN, C_PAD),
        ],
        out_specs=_row_spec(C_PAD),
        out_shape=jax.ShapeDtypeStruct((N_NODES, C_PAD), jnp.float32),
    )(aggpa[0, :N_NODES], aggpa[1, :N_NODES],
      aggpb[0, :N_NODES], aggpb[1, :N_NODES], ha, hb, nb,
      W1[:HALF], W1[HALF:], b1.reshape(1, N_HIDDEN), w2p)

    agg2p = _agg64(gs, gd, h2, zeros64)

    b2p = jnp.concatenate(
        [b2, jnp.zeros((C_PAD - N_CLASSES,), jnp.float32)]).reshape(1, C_PAD)
    out48 = pl.pallas_call(
        _out_body,
        grid=(grid,),
        in_specs=[
            _row_spec(C_PAD), _row_spec(C_PAD), _row_spec(C_PAD),
            _row_spec(C_PAD), _full_spec(1, C_PAD),
        ],
        out_specs=_row_spec(C_PAD),
        out_shape=jax.ShapeDtypeStruct((N_NODES, C_PAD), jnp.float32),
    )(agg2p[0, :N_NODES], agg2p[1, :N_NODES], h2, nb[:, :C_PAD], b2p)

    return out48[:, :N_CLASSES]
